# baseline (speedup 1.0000x reference)
import functools

import jax
import jax.numpy as jnp
from jax import lax
from jax.experimental import pallas as pl
from jax.experimental.pallas import tpu as pltpu
from jax.experimental.pallas import tpu_sc as plsc

N = 10000
D = 128
H = 32
E = 320000

NPAD = 10240
NW = 32
K = 128
NBATCH = 79
NBATCH1 = 157
EPAD1 = 16 * NBATCH1 * K
EPW = NBATCH * K
EPAD = NW * EPW
ROWS_PER_TILE = NPAD // 16
BM = 1024


@functools.lru_cache(maxsize=None)
def _make_segsum(C):
    mesh = plsc.VectorSubcoreMesh(core_axis_name="c", subcore_axis_name="s")

    @functools.partial(
        pl.kernel,
        out_type=jax.ShapeDtypeStruct((2, NPAD, C), jnp.float32),
        mesh=mesh,
        scratch_types=[
            pltpu.VMEM((NBATCH, K), jnp.int32),
            pltpu.VMEM((NBATCH, K), jnp.int32),
            pltpu.VMEM((2, K, C), jnp.float32),
            pltpu.VMEM_SHARED((NPAD, C), jnp.float32),
            pltpu.SemaphoreType.DMA,
            pltpu.SemaphoreType.DMA,
        ],
        compiler_params=pltpu.CompilerParams(use_tc_tiling_on_sc=False),
    )
    def seg(table_hbm, src_hbm, dst_hbm, zeros_hbm, out_hbm,
            src_v, dst_v, rows_v, acc_sh, sem0, sem1):
        c = lax.axis_index("c")
        s = lax.axis_index("s")
        w = c * 16 + s
        r0 = s * ROWS_PER_TILE
        pltpu.sync_copy(zeros_hbm.at[pl.ds(r0, ROWS_PER_TILE)],
                        acc_sh.at[pl.ds(r0, ROWS_PER_TILE)])
        pltpu.sync_copy(src_hbm.at[w], src_v)
        pltpu.sync_copy(dst_hbm.at[w], dst_v)
        plsc.subcore_barrier()

        pltpu.async_copy(table_hbm.at[src_v.at[0]], rows_v.at[0], sem0)

        def body2(i, _):
            j0 = i * 2

            @pl.when(j0 + 1 < NBATCH)
            def _g1():
                pltpu.async_copy(table_hbm.at[src_v.at[j0 + 1]],
                                 rows_v.at[1], sem1)
            pltpu.make_async_copy(table_hbm.at[src_v.at[j0]],
                                  rows_v.at[0], sem0).wait()
            pltpu.sync_copy(rows_v.at[0], acc_sh.at[dst_v.at[j0]], add=True)

            @pl.when(j0 + 2 < NBATCH)
            def _g2():
                pltpu.async_copy(table_hbm.at[src_v.at[j0 + 2]],
                                 rows_v.at[0], sem0)

            @pl.when(j0 + 1 < NBATCH)
            def _s2():
                pltpu.make_async_copy(table_hbm.at[src_v.at[j0 + 1]],
                                      rows_v.at[1], sem1).wait()
                pltpu.sync_copy(rows_v.at[1], acc_sh.at[dst_v.at[j0 + 1]],
                                add=True)
            return 0

        lax.fori_loop(0, (NBATCH + 1) // 2, body2, 0)
        plsc.subcore_barrier()
        pltpu.sync_copy(acc_sh.at[pl.ds(r0, ROWS_PER_TILE)],
                        out_hbm.at[c, pl.ds(r0, ROWS_PER_TILE)])

    return seg


@functools.lru_cache(maxsize=None)
def _make_segsum_colsplit():
    CH = D // 2
    mesh = plsc.VectorSubcoreMesh(core_axis_name="c", subcore_axis_name="s")

    @functools.partial(
        pl.kernel,
        out_type=jax.ShapeDtypeStruct((2, NPAD, CH), jnp.float32),
        mesh=mesh,
        scratch_types=[
            pltpu.VMEM((NBATCH1, K), jnp.int32),
            pltpu.VMEM((NBATCH1, K), jnp.int32),
            pltpu.VMEM((2, K, CH), jnp.float32),
            pltpu.VMEM_SHARED((NPAD, CH), jnp.float32),
            pltpu.SemaphoreType.DMA,
            pltpu.SemaphoreType.DMA,
        ],
        compiler_params=pltpu.CompilerParams(use_tc_tiling_on_sc=False),
    )
    def seg(tlo_hbm, thi_hbm, src_hbm, dst_hbm, zeros_hbm, out_hbm,
            src_v, dst_v, rows_v, acc_sh, sem0, sem1):
        c = lax.axis_index("c")
        s = lax.axis_index("s")
        r0 = s * ROWS_PER_TILE
        pltpu.sync_copy(zeros_hbm.at[pl.ds(r0, ROWS_PER_TILE)],
                        acc_sh.at[pl.ds(r0, ROWS_PER_TILE)])
        pltpu.sync_copy(src_hbm.at[s], src_v)
        pltpu.sync_copy(dst_hbm.at[s], dst_v)
        plsc.subcore_barrier()

        def run(table_hbm):
            pltpu.async_copy(table_hbm.at[src_v.at[0]], rows_v.at[0], sem0)

            def body2(i, _):
                j0 = i * 2

                @pl.when(j0 + 1 < NBATCH1)
                def _g1():
                    pltpu.async_copy(table_hbm.at[src_v.at[j0 + 1]],
                                     rows_v.at[1], sem1)
                pltpu.make_async_copy(table_hbm.at[src_v.at[j0]],
                                      rows_v.at[0], sem0).wait()
                pltpu.sync_copy(rows_v.at[0], acc_sh.at[dst_v.at[j0]],
                                add=True)

                @pl.when(j0 + 2 < NBATCH1)
                def _g2():
                    pltpu.async_copy(table_hbm.at[src_v.at[j0 + 2]],
                                     rows_v.at[0], sem0)

                @pl.when(j0 + 1 < NBATCH1)
                def _s2():
                    pltpu.make_async_copy(table_hbm.at[src_v.at[j0 + 1]],
                                          rows_v.at[1], sem1).wait()
                    pltpu.sync_copy(rows_v.at[1], acc_sh.at[dst_v.at[j0 + 1]],
                                    add=True)
                return 0

            lax.fori_loop(0, (NBATCH1 + 1) // 2, body2, 0)

        @pl.when(c == 0)
        def _lo():
            run(tlo_hbm)

        @pl.when(c == 1)
        def _hi():
            run(thi_hbm)

        plsc.subcore_barrier()
        pltpu.sync_copy(acc_sh.at[pl.ds(r0, ROWS_PER_TILE)],
                        out_hbm.at[c, pl.ds(r0, ROWS_PER_TILE)])

    return seg


def _dot(a, b):
    return jnp.dot(a, b, preferred_element_type=jnp.float32)


def _mlp(x_in, sa, sb, ba, Wa, bb, Wb):
    CI = x_in.shape[1]

    def body(x_ref, sa_ref, sb_ref, ba_ref, wa_ref, bb_ref, wb_ref, o_ref):
        agg = x_ref[...] + sa_ref[...] + sb_ref[...]
        h = jnp.maximum(_dot(agg, wa_ref[...]) + ba_ref[...], 0.0)
        o_ref[...] = _dot(h, wb_ref[...]) + bb_ref[...]

    return pl.pallas_call(
        body,
        grid=(NPAD // BM,),
        in_specs=[pl.BlockSpec((BM, CI), lambda i: (i, 0)),
                  pl.BlockSpec((BM, CI), lambda i: (i, 0)),
                  pl.BlockSpec((BM, CI), lambda i: (i, 0)),
                  pl.BlockSpec((1, H), lambda i: (0, 0)),
                  pl.BlockSpec((CI, H), lambda i: (0, 0)),
                  pl.BlockSpec((1, H), lambda i: (0, 0)),
                  pl.BlockSpec((H, H), lambda i: (0, 0))],
        out_specs=pl.BlockSpec((BM, H), lambda i: (i, 0)),
        out_shape=jax.ShapeDtypeStruct((NPAD, H), jnp.float32),
    )(x_in, sa, sb, ba, Wa, bb, Wb)


def _mlp1(x_in, slo, shi, ba, Wa, bb, Wb):
    CH = D // 2

    def body(x_ref, lo_ref, hi_ref, ba_ref, wa_ref, bb_ref, wb_ref, o_ref):
        agg = x_ref[...] + jnp.concatenate([lo_ref[...], hi_ref[...]], axis=1)
        h = jnp.maximum(_dot(agg, wa_ref[...]) + ba_ref[...], 0.0)
        o_ref[...] = _dot(h, wb_ref[...]) + bb_ref[...]

    return pl.pallas_call(
        body,
        grid=(NPAD // BM,),
        in_specs=[pl.BlockSpec((BM, D), lambda i: (i, 0)),
                  pl.BlockSpec((BM, CH), lambda i: (i, 0)),
                  pl.BlockSpec((BM, CH), lambda i: (i, 0)),
                  pl.BlockSpec((1, H), lambda i: (0, 0)),
                  pl.BlockSpec((D, H), lambda i: (0, 0)),
                  pl.BlockSpec((1, H), lambda i: (0, 0)),
                  pl.BlockSpec((H, H), lambda i: (0, 0))],
        out_specs=pl.BlockSpec((BM, H), lambda i: (i, 0)),
        out_shape=jax.ShapeDtypeStruct((NPAD, H), jnp.float32),
    )(x_in, slo, shi, ba, Wa, bb, Wb)


def _final(h2, sa, sb, W3, b3):
    def body(h_ref, sa_ref, sb_ref, w_ref, b_ref, o_ref):
        agg = h_ref[...] + sa_ref[...] + sb_ref[...]
        o_ref[...] = _dot(agg, w_ref[...]) + b_ref[...]
    return pl.pallas_call(
        body,
        grid=(NPAD // BM,),
        in_specs=[pl.BlockSpec((BM, H), lambda i: (i, 0)),
                  pl.BlockSpec((BM, H), lambda i: (i, 0)),
                  pl.BlockSpec((BM, H), lambda i: (i, 0)),
                  pl.BlockSpec((H, 1), lambda i: (0, 0)),
                  pl.BlockSpec((1, 1), lambda i: (0, 0))],
        out_specs=pl.BlockSpec((BM, 1), lambda i: (i, 0)),
        out_shape=jax.ShapeDtypeStruct((NPAD, 1), jnp.float32),
    )(h2, sa, sb, W3, b3)


def kernel(x, edge_index, W1a, b1a, W1b, b1b, W2a, b2a, W2b, b2b, W3, b3):
    src = edge_index[0]
    dst = edge_index[1]
    pad = EPAD - E
    src3 = jnp.concatenate(
        [src, jnp.zeros((pad,), jnp.int32)]).reshape(NW, NBATCH, K)
    dst3 = jnp.concatenate(
        [dst, jnp.full((pad,), N, jnp.int32)]).reshape(NW, NBATCH, K)
    pad1 = EPAD1 - E
    src4 = jnp.concatenate(
        [src, jnp.zeros((pad1,), jnp.int32)]).reshape(16, NBATCH1, K)
    dst4 = jnp.concatenate(
        [dst, jnp.full((pad1,), N, jnp.int32)]).reshape(16, NBATCH1, K)

    x_pad = jnp.pad(x, ((0, NPAD - N), (0, 0)))
    zeros64 = jnp.zeros((NPAD, D // 2), jnp.float32)
    zeros32 = jnp.zeros((NPAD, H), jnp.float32)

    s1 = _make_segsum_colsplit()(
        x_pad[:, :D // 2], x_pad[:, D // 2:], src4, dst4, zeros64)
    h1 = _mlp1(x_pad, s1[0], s1[1], b1a.reshape(1, H), W1a,
               b1b.reshape(1, H), W1b)
    s2 = _make_segsum(H)(h1, src3, dst3, zeros32)
    h2 = _mlp(h1, s2[0], s2[1], b2a.reshape(1, H), W2a,
              b2b.reshape(1, H), W2b)
    s3 = _make_segsum(H)(h2, src3, dst3, zeros32)
    out = _final(h2, s3[0], s3[1], W3, b3.reshape(1, 1))
    return out[:N]
